# Initial kernel scaffold; baseline (speedup 1.0000x reference)
import jax
import jax.numpy as jnp
from jax.experimental import pallas as pl


def kernel(inputs, edge_index, W, b):
    raise NotImplementedError("write your pallas kernel here")


# speedup vs baseline: 5.4838x; 5.4838x over previous
import functools

import jax
import jax.numpy as jnp
from jax import lax
from jax.experimental import pallas as pl
from jax.experimental.pallas import tpu as pltpu
from jax.experimental.pallas import tpu_sc as plsc

N_NODES = 10000
N_EDGES = 320000
D = 128

NC = 2
NS = 16
NW = NC * NS
EDGES_PER_TILE = N_EDGES // NW
CHUNK = 80
CHUNKS_PER_TILE = EDGES_PER_TILE // CHUNK
N_PAD = 10240
NODES_PER_TILE = N_PAD // NS


def _make_sc_scatter():
    mesh = plsc.VectorSubcoreMesh(core_axis_name="c", subcore_axis_name="s")

    @functools.partial(
        pl.kernel,
        mesh=mesh,
        out_type=jax.ShapeDtypeStruct((NC, N_PAD, D), jnp.float32),
        scratch_types=[
            pltpu.VMEM((CHUNK,), jnp.int32),
            pltpu.VMEM((CHUNK,), jnp.int32),
            pltpu.VMEM((CHUNK, D), jnp.float32),
            pltpu.VMEM_SHARED((N_PAD, D), jnp.float32),
            pltpu.SemaphoreType.DMA,
        ],
    )
    def sc_scatter(src_hbm, dst_hbm, x_hbm, zeros_hbm, out_hbm,
                   src_v, dst_v, rows_v, h_sh, sem):
        cid = lax.axis_index("c")
        sid = lax.axis_index("s")
        wid = sid * NC + cid

        row0 = sid * NODES_PER_TILE
        pltpu.sync_copy(zeros_hbm.at[pl.ds(row0, NODES_PER_TILE)],
                        h_sh.at[pl.ds(row0, NODES_PER_TILE)])
        plsc.subcore_barrier()

        tile_base = wid * EDGES_PER_TILE

        def chunk_body(i, _):
            base = tile_base + i * CHUNK
            pltpu.sync_copy(src_hbm.at[pl.ds(base, CHUNK)], src_v)
            pltpu.sync_copy(dst_hbm.at[pl.ds(base, CHUNK)], dst_v)
            pltpu.async_copy(x_hbm.at[src_v], rows_v, sem).wait()
            pltpu.sync_copy(rows_v, h_sh.at[dst_v], add=True)
            return _

        lax.fori_loop(0, CHUNKS_PER_TILE, chunk_body, 0)
        plsc.subcore_barrier()

        pltpu.sync_copy(h_sh.at[pl.ds(row0, NODES_PER_TILE)],
                        out_hbm.at[cid, pl.ds(row0, NODES_PER_TILE)])

    return sc_scatter


_sc_scatter = _make_sc_scatter()


def _tc_linear_body(h_ref, wt_ref, b_ref, o_ref):
    z = h_ref[0] + h_ref[1]
    acc = jnp.dot(z, wt_ref[...], preferred_element_type=jnp.float32)
    o_ref[...] = jnp.maximum(acc + b_ref[...], 0.0)


ROW_BLK = 1000


def _tc_linear(h, wt, b2):
    return pl.pallas_call(
        _tc_linear_body,
        grid=(N_NODES // ROW_BLK,),
        in_specs=[
            pl.BlockSpec((NC, ROW_BLK, D), lambda i: (0, i, 0)),
            pl.BlockSpec((D, D), lambda i: (0, 0)),
            pl.BlockSpec((1, D), lambda i: (0, 0)),
        ],
        out_specs=pl.BlockSpec((ROW_BLK, D), lambda i: (i, 0)),
        out_shape=jax.ShapeDtypeStruct((N_NODES, D), jnp.float32),
    )(h, wt, b2)


def kernel(inputs, edge_index, W, b):
    src = edge_index[0].astype(jnp.int32)
    dst = edge_index[1].astype(jnp.int32)
    zeros = jnp.zeros((N_PAD, D), jnp.float32)
    h = _sc_scatter(src, dst, inputs, zeros)
    return _tc_linear(h, W.T, b.reshape(1, D))
